# Initial kernel scaffold; baseline (speedup 1.0000x reference)
import jax
import jax.numpy as jnp
from jax.experimental import pallas as pl


def kernel(x, params):
    raise NotImplementedError("write your pallas kernel here")


# speedup vs baseline: 3.1216x; 3.1216x over previous
import jax
import jax.numpy as jnp
from jax import lax
from jax.experimental import pallas as pl
from jax.experimental.pallas import tpu as pltpu

f32 = jnp.float32
D = 192
HEADS = 12
E = 6
TOPK = 3
B, H, W = 2, 224, 224
TH = 16
NT = H // TH
RB = TH // 8
NPIX = float(H * W)


def _padc(x, d):
    r, wc, c = x.shape
    z = jnp.zeros((r, d, c), x.dtype)
    return jnp.concatenate([z, x, z], axis=1)


def _dw(x, w, b, d):
    r = x.shape[0]
    xp = _padc(x, d)
    acc = None
    for ky in range(3):
        for kx in range(3):
            s = xp[ky * d:ky * d + r - 2 * d, kx * d:kx * d + 224, :]
            t = s * w[ky, kx][None, None, :]
            acc = t if acc is None else acc + t
    if b is not None:
        acc = acc + b[None]
    return acc


def _c3(x, w, b):
    r, _, cin = x.shape
    cout = w.shape[3]
    xp = _padc(x, 1)
    m = (r - 2) * 224
    acc = jnp.zeros((m, cout), f32)
    for ky in range(3):
        for kx in range(3):
            s = xp[ky:ky + r - 2, kx:kx + 224, :].reshape(m, cin)
            acc = acc + jnp.dot(s, w[ky, kx], preferred_element_type=f32)
    b2 = acc.reshape(r - 2, 224, cout)
    if b is not None:
        b2 = b2 + b[None]
    return b2


def _assemble(m_ref, t_ref, b_ref, i, h):
    t = jnp.where(i > 0, t_ref[0, 8 - h:8, :, :], 0.0)
    bm = jnp.where(i < NT - 1, b_ref[0, 0:h, :, :], 0.0)
    return jnp.concatenate([t, m_ref[0], bm], axis=0)


def _maskrows(v, start):
    rows = lax.broadcasted_iota(jnp.int32, (v.shape[0], 1, 1), 0) + start
    return jnp.where((rows >= 0) & (rows < H), v, 0.0)


def _mspec(c):
    return pl.BlockSpec((1, TH, 224, c), lambda b, i: (b, i, 0, 0))


def _tspec(c):
    return pl.BlockSpec((1, 8, 224, c),
                        lambda b, i: (b, jnp.maximum(i * RB - 1, 0), 0, 0))


def _bspec(c):
    return pl.BlockSpec((1, 8, 224, c),
                        lambda b, i: (b, jnp.minimum(i * RB + RB, H // 8 - 1),
                                      0, 0))


def _wspec2(shape):
    return pl.BlockSpec(shape, lambda b, i: (0,) * len(shape))


def _k1(x_ref, o_ref):
    b = pl.program_id(0)
    i = pl.program_id(1)
    x2 = x_ref[0].reshape(TH * 224, D)
    s = jnp.sum(x2, axis=0, keepdims=True)
    ss = jnp.sum(x2 * x2, axis=0, keepdims=True)
    st = jnp.concatenate([s, ss], axis=0)

    @pl.when((b == 0) & (i == 0))
    def _():
        o_ref[...] = st

    @pl.when((b > 0) | (i > 0))
    def _():
        o_ref[...] = o_ref[...] + st


def _k2(xm, xt, xb, bns, bng, bnb, wq, bq, dww, dwb, cw, cb,
        a_o, xh_o, v_o, g_o, n_o):
    i = pl.program_id(1)
    bn = bns[...]
    mean = bn[0:1] / (B * NPIX)
    var = bn[1:2] / (B * NPIX) - mean * mean
    scale = bng[...] / jnp.sqrt(var + 1e-5)
    shift = bnb[...] - mean * scale
    rows = _assemble(xm, xt, xb, i, 1)
    a_ext = _maskrows(rows * scale[None] + shift[None], i * TH - 1)
    a_o[0] = a_ext[1:TH + 1]

    qkv = jnp.dot(a_ext.reshape((TH + 2) * 224, D), wq[...],
                  preferred_element_type=f32) + bq[...]
    qkv3 = _maskrows(qkv.reshape(TH + 2, 224, 3 * D), i * TH - 1)
    qkvd = _dw(qkv3, dww[...], dwb[0], 1)
    v_o[0] = qkvd[:, :, 2 * D:3 * D]
    q2 = qkvd[:, :, 0:D].reshape(TH * 224, D)
    k2 = qkvd[:, :, D:2 * D].reshape(TH * 224, D)
    g = lax.dot_general(q2, k2, (((0,), (0,)), ((), ())),
                        preferred_element_type=f32)
    qs = jnp.sum(q2 * q2, axis=0, keepdims=True)
    ks = jnp.sum(k2 * k2, axis=0, keepdims=True)
    nrm = jnp.concatenate([qs, ks], axis=0)

    dwa = _dw(a_ext, cw[...], cb[0], 1)
    xh_o[0] = a_ext[1:TH + 1] + dwa

    @pl.when(i == 0)
    def _():
        g_o[0] = g
        n_o[0] = nrm

    @pl.when(i > 0)
    def _():
        g_o[0] = g_o[0] + g
        n_o[0] = n_o[0] + nrm


def _k3(g_ref, n_ref, t_ref, wp_ref, m_o, a_scr):
    g = g_ref[0]
    nr = n_ref[0]
    qn = jnp.maximum(jnp.sqrt(nr[0:1, :]), 1e-12)
    kn = jnp.maximum(jnp.sqrt(nr[1:2, :]), 1e-12)
    arow = t_ref[...] / qn
    ones = jnp.ones((1, D), f32)
    amat = lax.dot_general(arow, ones, (((0,), (0,)), ((), ())),
                           preferred_element_type=f32)
    z = g * amat / kn
    a_scr[...] = jnp.zeros((D, D), f32)
    for hh in range(HEADS):
        sl = slice(16 * hh, 16 * hh + 16)
        blk = z[sl, sl]
        mx = jnp.max(blk, axis=1, keepdims=True)
        e = jnp.exp(blk - mx)
        at = e / jnp.sum(e, axis=1, keepdims=True)
        a_scr[sl, sl] = at
    m_o[0] = lax.dot_general(a_scr[...], wp_ref[...],
                             (((0,), (0,)), ((), ())),
                             preferred_element_type=f32)


def _k4(a_ref, xh_ref, v_ref, m_ref, bp_ref, xb_o, df_o, st_o):
    i = pl.program_id(1)
    ao = jnp.dot(v_ref[0].reshape(TH * 224, D), m_ref[0],
                 preferred_element_type=f32) + bp_ref[...]
    xb = a_ref[0] + ao.reshape(TH, 224, D)
    diff = xb - xh_ref[0]
    xb_o[0] = xb
    df_o[0] = diff
    d2 = diff.reshape(TH * 224, D)
    s = jnp.sum(d2, axis=0, keepdims=True)
    mx = jnp.max(d2, axis=0, keepdims=True)
    mn = jnp.min(d2, axis=0, keepdims=True)

    @pl.when(i == 0)
    def _():
        st_o[0] = jnp.concatenate([s, mx, mn], axis=0)

    @pl.when(i > 0)
    def _():
        cur = st_o[0]
        st_o[0] = jnp.concatenate([cur[0:1] + s,
                                   jnp.maximum(cur[1:2], mx),
                                   jnp.minimum(cur[2:3], mn)], axis=0)


def _gate_math(pooled, f0, b0, f1, b1):
    h = jnp.dot(pooled, f0, preferred_element_type=f32) + b0
    noise = jax.nn.softplus(h)
    hh = jnp.dot(pooled, f1, preferred_element_type=f32) + b1
    hh = jnp.where(hh >= 0, hh, 0.2 * hh)
    nm = jnp.mean(noise, axis=1, keepdims=True)
    std = jnp.sqrt(jnp.sum((noise - nm) ** 2, axis=1, keepdims=True)
                   / (E - 1))
    s = hh + (noise - nm) / std
    col = lax.broadcasted_iota(jnp.int32, (B, E), 1)
    cnt = jnp.zeros((B, E), jnp.int32)
    for j in range(E):
        sj = s[:, j:j + 1]
        cnt = cnt + jnp.where(sj > s, 1, 0)
        cnt = cnt + jnp.where((sj == s) & (col > j), 1, 0)
    mask = cnt < TOPK
    hmax = jnp.max(jnp.where(mask, hh, -1e30), axis=1, keepdims=True)
    ex = jnp.where(mask, jnp.exp(hh - hmax), 0.0)
    return ex / jnp.sum(ex, axis=1, keepdims=True)


def _k5(st_ref, f0d, b0d, f1d, b1d, f0t, b0t, f1t, b1t, cof_o):
    st = st_ref[...]
    mean = st[:, 0, :] / NPIX
    mx = st[:, 1, :]
    mn = st[:, 2, :]
    cof_o[0] = _gate_math(mx + mean, f0d[...], b0d[...], f1d[...], b1d[...])
    cof_o[1] = _gate_math(-mn - mean, f0t[...], b0t[...], f1t[...], b1t[...])


def _k6(dm, dt, db, cof, w1, b1, w2, b2, out_o, gs_o, *, sign, dil):
    b = pl.program_id(0)
    i = pl.program_id(1)
    rows = _assemble(dm, dt, db, i, 2 * dil) * sign
    out_o[0] = jnp.zeros((TH, 224, D), f32)
    for e in range(E):
        wgt = cof[b, e]

        @pl.when(wgt > 0.0)
        def _(e=e, wgt=wgt):
            h1 = _dw(rows, w1[e], b1[e], dil)
            h1 = jnp.maximum(h1, 0.0)
            h1 = _maskrows(h1, i * TH - dil)
            h2 = _dw(h1, w2[e], b2[e], dil)
            out_o[0] = out_o[0] + wgt * h2

    ts = jnp.sum(out_o[0].reshape(TH * 224, D), axis=0, keepdims=True)

    @pl.when(i == 0)
    def _():
        gs_o[0] = ts

    @pl.when(i > 0)
    def _():
        gs_o[0] = gs_o[0] + ts


def _k7(gc_ref, gt_ref, w1, b1, w2, b2, w1b, b1b, w2b, b2b, eca_o, eta_o):
    gc = gc_ref[...].reshape(B, D) / NPIX
    gt = gt_ref[...].reshape(B, D) / NPIX
    h = jnp.maximum(jnp.dot(gc, w1[...], preferred_element_type=f32)
                    + b1[...], 0.0)
    o = jnp.dot(h, w2[...], preferred_element_type=f32) + b2[...]
    eca_o[...] = jax.nn.sigmoid(o).reshape(B, 1, D)
    h = jnp.maximum(jnp.dot(gt, w1b[...], preferred_element_type=f32)
                    + b1b[...], 0.0)
    o = jnp.dot(h, w2b[...], preferred_element_type=f32) + b2b[...]
    eta_o[...] = jax.nn.sigmoid(o).reshape(B, 1, D)


def _k8(em, et, eb, eca, w1, b1, w2, b2, w3, b3, w4, b4, w5, b5, w6, b6,
        out_o):
    i = pl.program_id(1)
    rows = _assemble(em, et, eb, i, 4)
    h1 = jnp.dot(rows.reshape((TH + 8) * 224, D), w1[...],
                 preferred_element_type=f32) + b1[...]
    h1 = _maskrows(h1.reshape(TH + 8, 224, D // 2), i * TH - 4)
    h2 = _maskrows(jnp.maximum(_c3(h1, w2[...], b2[...]), 0.0), i * TH - 3)
    h3 = _maskrows(jnp.maximum(_c3(h2, w3[...], b3[...]), 0.0), i * TH - 2)
    h4 = _maskrows(jnp.maximum(_c3(h3, w4[...], b4[...]), 0.0), i * TH - 1)
    h5 = _c3(h4, w5[...], b5[...])
    h6 = jnp.dot(h5.reshape(TH * 224, D // 2), w6[...],
                 preferred_element_type=f32) + b6[...]
    ect = jax.nn.sigmoid(h6.reshape(TH, 224, D))
    out_o[0] = em[0] * ect + (1.0 - ect) * eca[0]


def _k9(cm, ct, cb, tm, tt, tb, xb_ref, xh_ref,
        xw, xbi, yw, ybi, mw, mbi, nw, nbi, out_o):
    i = pl.program_id(1)
    rc = _assemble(cm, ct, cb, i, 1)
    rt = _assemble(tm, tt, tb, i, 1)
    xo = _dw(rc, xw[...], xbi[0], 1)
    yo = _dw(rc, yw[...], ybi[0], 1)
    mo = _dw(rt, mw[...], mbi[0], 1)
    no = _dw(rt, nw[...], nbi[0], 1)
    out_o[0] = xo * xb_ref[0] + yo + mo * xh_ref[0] + no


def _dwwt(w):
    return jnp.transpose(w[:, 0], (1, 2, 0))


def kernel(x, params):
    p = params
    xt = jnp.transpose(x, (0, 2, 3, 1))

    img = lambda c: jax.ShapeDtypeStruct((B, H, W, c), f32)
    r2 = lambda a: a.reshape(1, -1)

    bns = pl.pallas_call(
        _k1, grid=(B, NT),
        in_specs=[_mspec(D)],
        out_specs=pl.BlockSpec((2, D), lambda b, i: (0, 0)),
        out_shape=jax.ShapeDtypeStruct((2, D), f32),
    )(xt)

    wq = jnp.transpose(p['attn_qkv_w'][:, :, 0, 0])
    a_arr, xh, v, gram, nrm = pl.pallas_call(
        _k2, grid=(B, NT),
        in_specs=[_mspec(D), _tspec(D), _bspec(D),
                  _wspec2((2, D)), _wspec2((1, D)), _wspec2((1, D)),
                  _wspec2((D, 3 * D)), _wspec2((1, 3 * D)),
                  _wspec2((3, 3, 3 * D)), _wspec2((1, 3 * D)),
                  _wspec2((3, 3, D)), _wspec2((1, D))],
        out_specs=[_mspec(D), _mspec(D), _mspec(D),
                   pl.BlockSpec((1, D, D), lambda b, i: (b, 0, 0)),
                   pl.BlockSpec((1, 2, D), lambda b, i: (b, 0, 0))],
        out_shape=[img(D), img(D), img(D),
                   jax.ShapeDtypeStruct((B, D, D), f32),
                   jax.ShapeDtypeStruct((B, 2, D), f32)],
    )(xt, xt, xt, bns, r2(p['bn_g']), r2(p['bn_b']), wq,
      r2(p['attn_qkv_b']), _dwwt(p['attn_dw_w']), r2(p['attn_dw_b']),
      _dwwt(p['conv_w']), r2(p['conv_b']))

    trow = jnp.repeat(p['attn_temp'].reshape(HEADS), 16).reshape(1, D)
    wpt = jnp.transpose(p['attn_proj_w'][:, :, 0, 0])
    mproj = pl.pallas_call(
        _k3, grid=(B,),
        in_specs=[pl.BlockSpec((1, D, D), lambda b: (b, 0, 0)),
                  pl.BlockSpec((1, 2, D), lambda b: (b, 0, 0)),
                  pl.BlockSpec((1, D), lambda b: (0, 0)),
                  pl.BlockSpec((D, D), lambda b: (0, 0))],
        out_specs=pl.BlockSpec((1, D, D), lambda b: (b, 0, 0)),
        out_shape=jax.ShapeDtypeStruct((B, D, D), f32),
        scratch_shapes=[pltpu.VMEM((D, D), f32)],
    )(gram, nrm, trow, wpt)

    xb_arr, diff, stats = pl.pallas_call(
        _k4, grid=(B, NT),
        in_specs=[_mspec(D), _mspec(D), _mspec(D),
                  pl.BlockSpec((1, D, D), lambda b, i: (b, 0, 0)),
                  _wspec2((1, D))],
        out_specs=[_mspec(D), _mspec(D),
                   pl.BlockSpec((1, 3, D), lambda b, i: (b, 0, 0))],
        out_shape=[img(D), img(D), jax.ShapeDtypeStruct((B, 3, D), f32)],
    )(a_arr, xh, v, mproj, r2(p['attn_proj_b']))

    gspec = lambda s: pl.BlockSpec(s, lambda i: (0,) * len(s))
    cof = pl.pallas_call(
        _k5, grid=(1,),
        in_specs=[gspec((B, 3, D)),
                  gspec((D, E)), gspec((1, E)), gspec((D, E)), gspec((1, E)),
                  gspec((D, E)), gspec((1, E)), gspec((D, E)), gspec((1, E))],
        out_specs=gspec((2, B, E)),
        out_shape=jax.ShapeDtypeStruct((2, B, E), f32),
    )(stats,
      jnp.transpose(p['dec_fc0_w']), r2(p['dec_fc0_b']),
      jnp.transpose(p['dec_fc1_w']), r2(p['dec_fc1_b']),
      jnp.transpose(p['det_fc0_w']), r2(p['det_fc0_b']),
      jnp.transpose(p['det_fc1_w']), r2(p['det_fc1_b']))

    def mofe(g, sign, dil, pre):
        import functools
        kfn = functools.partial(_k6, sign=sign, dil=dil)
        return pl.pallas_call(
            kfn, grid=(B, NT),
            in_specs=[_mspec(D), _tspec(D), _bspec(D),
                      pl.BlockSpec(memory_space=pltpu.SMEM),
                      _wspec2((E, 3, 3, D)), _wspec2((E, 1, D)),
                      _wspec2((E, 3, 3, D)), _wspec2((E, 1, D))],
            out_specs=[_mspec(D),
                       pl.BlockSpec((1, 1, D), lambda b, i: (b, 0, 0))],
            out_shape=[img(D), jax.ShapeDtypeStruct((B, 1, D), f32)],
        )(diff, diff, diff, cof[g],
          jnp.transpose(p[pre + '_w1'][:, :, 0], (0, 2, 3, 1)),
          p[pre + '_b1'][:, None, :],
          jnp.transpose(p[pre + '_w2'][:, :, 0], (0, 2, 3, 1)),
          p[pre + '_b2'][:, None, :])

    exp_c, gs_c = mofe(0, 1.0, 1, 'dec')
    exp_t, gs_t = mofe(1, -1.0, 2, 'det')

    eca, eta = pl.pallas_call(
        _k7, grid=(1,),
        in_specs=[gspec((B, 1, D)), gspec((B, 1, D)),
                  gspec((D, 2 * D)), gspec((1, 2 * D)),
                  gspec((2 * D, D)), gspec((1, D)),
                  gspec((D, 2 * D)), gspec((1, 2 * D)),
                  gspec((2 * D, D)), gspec((1, D))],
        out_specs=[gspec((B, 1, D)), gspec((B, 1, D))],
        out_shape=[jax.ShapeDtypeStruct((B, 1, D), f32),
                   jax.ShapeDtypeStruct((B, 1, D), f32)],
    )(gs_c, gs_t,
      jnp.transpose(p['mlp_w1']), r2(p['mlp_b1']),
      jnp.transpose(p['mlp_w2']), r2(p['mlp_b2']),
      jnp.transpose(p['mlp1_w1']), r2(p['mlp1_b1']),
      jnp.transpose(p['mlp1_w2']), r2(p['mlp1_b2']))

    def unet(expa, gate, pre):
        cwt = lambda w: jnp.transpose(w, (2, 3, 1, 0))
        return pl.pallas_call(
            _k8, grid=(B, NT),
            in_specs=[_mspec(D), _tspec(D), _bspec(D),
                      pl.BlockSpec((1, 1, D), lambda b, i: (b, 0, 0)),
                      _wspec2((D, D // 2)), _wspec2((1, D // 2)),
                      _wspec2((3, 3, D // 2, D // 4)), _wspec2((1, D // 4)),
                      _wspec2((3, 3, D // 4, D // 8)), _wspec2((1, D // 8)),
                      _wspec2((3, 3, D // 8, D // 4)), _wspec2((1, D // 4)),
                      _wspec2((3, 3, D // 4, D // 2)), _wspec2((1, D // 2)),
                      _wspec2((D // 2, D)), _wspec2((1, D))],
            out_specs=_mspec(D),
            out_shape=img(D),
        )(expa, expa, expa, gate,
          jnp.transpose(p[pre + '_w1'][:, :, 0, 0]), r2(p[pre + '_b1']),
          cwt(p[pre + '_w2']), r2(p[pre + '_b2']),
          cwt(p[pre + '_w3']), r2(p[pre + '_b3']),
          cwt(p[pre + '_w4']), r2(p[pre + '_b4']),
          cwt(p[pre + '_w5']), r2(p[pre + '_b5']),
          jnp.transpose(p[pre + '_w6'][:, :, 0, 0]), r2(p[pre + '_b6']))

    ecp = unet(exp_c, eca, 'u')
    etp = unet(exp_t, eta, 'u1')

    out = pl.pallas_call(
        _k9, grid=(B, NT),
        in_specs=[_mspec(D), _tspec(D), _bspec(D),
                  _mspec(D), _tspec(D), _bspec(D),
                  _mspec(D), _mspec(D),
                  _wspec2((3, 3, D)), _wspec2((1, D)),
                  _wspec2((3, 3, D)), _wspec2((1, D)),
                  _wspec2((3, 3, D)), _wspec2((1, D)),
                  _wspec2((3, 3, D)), _wspec2((1, D))],
        out_specs=_mspec(D),
        out_shape=img(D),
    )(ecp, ecp, ecp, etp, etp, etp, xb_arr, xh,
      _dwwt(p['X_w']), r2(p['X_b']), _dwwt(p['Y_w']), r2(p['Y_b']),
      _dwwt(p['M_w']), r2(p['M_b']), _dwwt(p['N_w']), r2(p['N_b']))

    return jnp.transpose(out, (0, 3, 1, 2))


# speedup vs baseline: 3.1307x; 1.0029x over previous
import jax
import jax.numpy as jnp
from jax import lax
from jax.experimental import pallas as pl
from jax.experimental.pallas import tpu as pltpu

f32 = jnp.float32
D = 192
HEADS = 12
E = 6
TOPK = 3
B, H, W = 2, 224, 224
TH = 16
NT = H // TH
RB = TH // 8
NPIX = float(H * W)


def _mm(a, b):
    return jnp.dot(a.astype(jnp.bfloat16), b.astype(jnp.bfloat16),
                   preferred_element_type=f32)


def _padc(x, d):
    r, wc, c = x.shape
    z = jnp.zeros((r, d, c), x.dtype)
    return jnp.concatenate([z, x, z], axis=1)


def _dw(x, w, b, d):
    r = x.shape[0]
    xp = _padc(x, d)
    acc = None
    for ky in range(3):
        for kx in range(3):
            s = xp[ky * d:ky * d + r - 2 * d, kx * d:kx * d + 224, :]
            t = s * w[ky, kx][None, None, :]
            acc = t if acc is None else acc + t
    if b is not None:
        acc = acc + b[None]
    return acc


def _c3(x, w, b):
    r, _, cin = x.shape
    cout = w.shape[3]
    xp = _padc(x, 1)
    m = (r - 2) * 224
    acc = jnp.zeros((m, cout), f32)
    for ky in range(3):
        for kx in range(3):
            s = xp[ky:ky + r - 2, kx:kx + 224, :].reshape(m, cin)
            acc = acc + _mm(s, w[ky, kx])
    b2 = acc.reshape(r - 2, 224, cout)
    if b is not None:
        b2 = b2 + b[None]
    return b2


def _assemble(m_ref, t_ref, b_ref, i, h):
    t = jnp.where(i > 0, t_ref[0, 8 - h:8, :, :], 0.0)
    bm = jnp.where(i < NT - 1, b_ref[0, 0:h, :, :], 0.0)
    return jnp.concatenate([t, m_ref[0], bm], axis=0)


def _maskrows(v, start):
    rows = lax.broadcasted_iota(jnp.int32, (v.shape[0], 1, 1), 0) + start
    return jnp.where((rows >= 0) & (rows < H), v, 0.0)


def _mspec(c):
    return pl.BlockSpec((1, TH, 224, c), lambda b, i: (b, i, 0, 0))


def _tspec(c):
    return pl.BlockSpec((1, 8, 224, c),
                        lambda b, i: (b, jnp.maximum(i * RB - 1, 0), 0, 0))


def _bspec(c):
    return pl.BlockSpec((1, 8, 224, c),
                        lambda b, i: (b, jnp.minimum(i * RB + RB, H // 8 - 1),
                                      0, 0))


def _wspec2(shape):
    return pl.BlockSpec(shape, lambda b, i: (0,) * len(shape))


def _k1(x_ref, o_ref):
    b = pl.program_id(0)
    i = pl.program_id(1)
    x2 = x_ref[0].reshape(TH * 224, D)
    s = jnp.sum(x2, axis=0, keepdims=True)
    ss = jnp.sum(x2 * x2, axis=0, keepdims=True)
    st = jnp.concatenate([s, ss], axis=0)

    @pl.when((b == 0) & (i == 0))
    def _():
        o_ref[...] = st

    @pl.when((b > 0) | (i > 0))
    def _():
        o_ref[...] = o_ref[...] + st


def _k2(xm, xt, xb, bns, bng, bnb, wq, bq, dww, dwb, cw, cb,
        a_o, xh_o, v_o, g_o, n_o):
    i = pl.program_id(1)
    bn = bns[...]
    mean = bn[0:1] / (B * NPIX)
    var = bn[1:2] / (B * NPIX) - mean * mean
    scale = bng[...] / jnp.sqrt(var + 1e-5)
    shift = bnb[...] - mean * scale
    rows = _assemble(xm, xt, xb, i, 1)
    a_ext = _maskrows(rows * scale[None] + shift[None], i * TH - 1)
    a_o[0] = a_ext[1:TH + 1]

    qkv = _mm(a_ext.reshape((TH + 2) * 224, D), wq[...]) + bq[...]
    qkv3 = _maskrows(qkv.reshape(TH + 2, 224, 3 * D), i * TH - 1)
    qkvd = _dw(qkv3, dww[...], dwb[0], 1)
    v_o[0] = qkvd[:, :, 2 * D:3 * D]
    q2 = qkvd[:, :, 0:D].reshape(TH * 224, D)
    k2 = qkvd[:, :, D:2 * D].reshape(TH * 224, D)
    g = lax.dot_general(q2, k2, (((0,), (0,)), ((), ())),
                        preferred_element_type=f32)
    qs = jnp.sum(q2 * q2, axis=0, keepdims=True)
    ks = jnp.sum(k2 * k2, axis=0, keepdims=True)
    nrm = jnp.concatenate([qs, ks], axis=0)

    dwa = _dw(a_ext, cw[...], cb[0], 1)
    xh_o[0] = a_ext[1:TH + 1] + dwa

    @pl.when(i == 0)
    def _():
        g_o[0] = g
        n_o[0] = nrm

    @pl.when(i > 0)
    def _():
        g_o[0] = g_o[0] + g
        n_o[0] = n_o[0] + nrm


def _k3(g_ref, n_ref, t_ref, wp_ref, m_o, a_scr):
    g = g_ref[0]
    nr = n_ref[0]
    qn = jnp.maximum(jnp.sqrt(nr[0:1, :]), 1e-12)
    kn = jnp.maximum(jnp.sqrt(nr[1:2, :]), 1e-12)
    arow = t_ref[...] / qn
    ones = jnp.ones((1, D), f32)
    amat = lax.dot_general(arow, ones, (((0,), (0,)), ((), ())),
                           preferred_element_type=f32)
    z = g * amat / kn
    a_scr[...] = jnp.zeros((D, D), f32)
    for hh in range(HEADS):
        sl = slice(16 * hh, 16 * hh + 16)
        blk = z[sl, sl]
        mx = jnp.max(blk, axis=1, keepdims=True)
        e = jnp.exp(blk - mx)
        at = e / jnp.sum(e, axis=1, keepdims=True)
        a_scr[sl, sl] = at
    m_o[0] = lax.dot_general(a_scr[...], wp_ref[...],
                             (((0,), (0,)), ((), ())),
                             preferred_element_type=f32)


def _k4(a_ref, xh_ref, v_ref, m_ref, bp_ref, xb_o, df_o, st_o):
    i = pl.program_id(1)
    ao = _mm(v_ref[0].reshape(TH * 224, D), m_ref[0]) + bp_ref[...]
    xb = a_ref[0] + ao.reshape(TH, 224, D)
    diff = xb - xh_ref[0]
    xb_o[0] = xb
    df_o[0] = diff
    d2 = diff.reshape(TH * 224, D)
    s = jnp.sum(d2, axis=0, keepdims=True)
    mx = jnp.max(d2, axis=0, keepdims=True)
    mn = jnp.min(d2, axis=0, keepdims=True)

    @pl.when(i == 0)
    def _():
        st_o[0] = jnp.concatenate([s, mx, mn], axis=0)

    @pl.when(i > 0)
    def _():
        cur = st_o[0]
        st_o[0] = jnp.concatenate([cur[0:1] + s,
                                   jnp.maximum(cur[1:2], mx),
                                   jnp.minimum(cur[2:3], mn)], axis=0)


def _gate_math(pooled, f0, b0, f1, b1):
    h = jnp.dot(pooled, f0, preferred_element_type=f32) + b0
    noise = jax.nn.softplus(h)
    hh = jnp.dot(pooled, f1, preferred_element_type=f32) + b1
    hh = jnp.where(hh >= 0, hh, 0.2 * hh)
    nm = jnp.mean(noise, axis=1, keepdims=True)
    std = jnp.sqrt(jnp.sum((noise - nm) ** 2, axis=1, keepdims=True)
                   / (E - 1))
    s = hh + (noise - nm) / std
    col = lax.broadcasted_iota(jnp.int32, (B, E), 1)
    cnt = jnp.zeros((B, E), jnp.int32)
    for j in range(E):
        sj = s[:, j:j + 1]
        cnt = cnt + jnp.where(sj > s, 1, 0)
        cnt = cnt + jnp.where((sj == s) & (col > j), 1, 0)
    mask = cnt < TOPK
    hmax = jnp.max(jnp.where(mask, hh, -1e30), axis=1, keepdims=True)
    ex = jnp.where(mask, jnp.exp(hh - hmax), 0.0)
    return ex / jnp.sum(ex, axis=1, keepdims=True)


def _k5(st_ref, f0d, b0d, f1d, b1d, f0t, b0t, f1t, b1t, cof_o):
    st = st_ref[...]
    mean = st[:, 0, :] / NPIX
    mx = st[:, 1, :]
    mn = st[:, 2, :]
    cof_o[0] = _gate_math(mx + mean, f0d[...], b0d[...], f1d[...], b1d[...])
    cof_o[1] = _gate_math(-mn - mean, f0t[...], b0t[...], f1t[...], b1t[...])


def _k6(dm, dt, db, cof, w1, b1, w2, b2, out_o, gs_o, *, sign, dil):
    b = pl.program_id(0)
    i = pl.program_id(1)
    rows = _assemble(dm, dt, db, i, 2 * dil) * sign
    out_o[0] = jnp.zeros((TH, 224, D), f32)
    for e in range(E):
        wgt = cof[b, e]

        @pl.when(wgt > 0.0)
        def _(e=e, wgt=wgt):
            h1 = _dw(rows, w1[e], b1[e], dil)
            h1 = jnp.maximum(h1, 0.0)
            h1 = _maskrows(h1, i * TH - dil)
            h2 = _dw(h1, w2[e], b2[e], dil)
            out_o[0] = out_o[0] + wgt * h2

    ts = jnp.sum(out_o[0].reshape(TH * 224, D), axis=0, keepdims=True)

    @pl.when(i == 0)
    def _():
        gs_o[0] = ts

    @pl.when(i > 0)
    def _():
        gs_o[0] = gs_o[0] + ts


def _k7(gc_ref, gt_ref, w1, b1, w2, b2, w1b, b1b, w2b, b2b, eca_o, eta_o):
    gc = gc_ref[...].reshape(B, D) / NPIX
    gt = gt_ref[...].reshape(B, D) / NPIX
    h = jnp.maximum(jnp.dot(gc, w1[...], preferred_element_type=f32)
                    + b1[...], 0.0)
    o = jnp.dot(h, w2[...], preferred_element_type=f32) + b2[...]
    eca_o[...] = jax.nn.sigmoid(o).reshape(B, 1, D)
    h = jnp.maximum(jnp.dot(gt, w1b[...], preferred_element_type=f32)
                    + b1b[...], 0.0)
    o = jnp.dot(h, w2b[...], preferred_element_type=f32) + b2b[...]
    eta_o[...] = jax.nn.sigmoid(o).reshape(B, 1, D)


def _k8(em, et, eb, eca, w1, b1, w2, b2, w3, b3, w4, b4, w5, b5, w6, b6,
        out_o):
    i = pl.program_id(1)
    rows = _assemble(em, et, eb, i, 4)
    h1 = _mm(rows.reshape((TH + 8) * 224, D), w1[...]) + b1[...]
    h1 = _maskrows(h1.reshape(TH + 8, 224, D // 2), i * TH - 4)
    h2 = _maskrows(jnp.maximum(_c3(h1, w2[...], b2[...]), 0.0), i * TH - 3)
    h3 = _maskrows(jnp.maximum(_c3(h2, w3[...], b3[...]), 0.0), i * TH - 2)
    h4 = _maskrows(jnp.maximum(_c3(h3, w4[...], b4[...]), 0.0), i * TH - 1)
    h5 = _c3(h4, w5[...], b5[...])
    h6 = _mm(h5.reshape(TH * 224, D // 2), w6[...]) + b6[...]
    ect = jax.nn.sigmoid(h6.reshape(TH, 224, D))
    out_o[0] = em[0] * ect + (1.0 - ect) * eca[0]


def _k9(cm, ct, cb, tm, tt, tb, xb_ref, xh_ref,
        xw, xbi, yw, ybi, mw, mbi, nw, nbi, out_o):
    i = pl.program_id(1)
    rc = _assemble(cm, ct, cb, i, 1)
    rt = _assemble(tm, tt, tb, i, 1)
    xo = _dw(rc, xw[...], xbi[0], 1)
    yo = _dw(rc, yw[...], ybi[0], 1)
    mo = _dw(rt, mw[...], mbi[0], 1)
    no = _dw(rt, nw[...], nbi[0], 1)
    out_o[0] = xo * xb_ref[0] + yo + mo * xh_ref[0] + no


def _dwwt(w):
    return jnp.transpose(w[:, 0], (1, 2, 0))


def kernel(x, params):
    p = params
    xt = jnp.transpose(x, (0, 2, 3, 1))

    img = lambda c: jax.ShapeDtypeStruct((B, H, W, c), f32)
    r2 = lambda a: a.reshape(1, -1)

    bns = pl.pallas_call(
        _k1, grid=(B, NT),
        in_specs=[_mspec(D)],
        out_specs=pl.BlockSpec((2, D), lambda b, i: (0, 0)),
        out_shape=jax.ShapeDtypeStruct((2, D), f32),
    )(xt)

    wq = jnp.transpose(p['attn_qkv_w'][:, :, 0, 0])
    a_arr, xh, v, gram, nrm = pl.pallas_call(
        _k2, grid=(B, NT),
        in_specs=[_mspec(D), _tspec(D), _bspec(D),
                  _wspec2((2, D)), _wspec2((1, D)), _wspec2((1, D)),
                  _wspec2((D, 3 * D)), _wspec2((1, 3 * D)),
                  _wspec2((3, 3, 3 * D)), _wspec2((1, 3 * D)),
                  _wspec2((3, 3, D)), _wspec2((1, D))],
        out_specs=[_mspec(D), _mspec(D), _mspec(D),
                   pl.BlockSpec((1, D, D), lambda b, i: (b, 0, 0)),
                   pl.BlockSpec((1, 2, D), lambda b, i: (b, 0, 0))],
        out_shape=[img(D), img(D), img(D),
                   jax.ShapeDtypeStruct((B, D, D), f32),
                   jax.ShapeDtypeStruct((B, 2, D), f32)],
    )(xt, xt, xt, bns, r2(p['bn_g']), r2(p['bn_b']), wq,
      r2(p['attn_qkv_b']), _dwwt(p['attn_dw_w']), r2(p['attn_dw_b']),
      _dwwt(p['conv_w']), r2(p['conv_b']))

    trow = jnp.repeat(p['attn_temp'].reshape(HEADS), 16).reshape(1, D)
    wpt = jnp.transpose(p['attn_proj_w'][:, :, 0, 0])
    mproj = pl.pallas_call(
        _k3, grid=(B,),
        in_specs=[pl.BlockSpec((1, D, D), lambda b: (b, 0, 0)),
                  pl.BlockSpec((1, 2, D), lambda b: (b, 0, 0)),
                  pl.BlockSpec((1, D), lambda b: (0, 0)),
                  pl.BlockSpec((D, D), lambda b: (0, 0))],
        out_specs=pl.BlockSpec((1, D, D), lambda b: (b, 0, 0)),
        out_shape=jax.ShapeDtypeStruct((B, D, D), f32),
        scratch_shapes=[pltpu.VMEM((D, D), f32)],
    )(gram, nrm, trow, wpt)

    xb_arr, diff, stats = pl.pallas_call(
        _k4, grid=(B, NT),
        in_specs=[_mspec(D), _mspec(D), _mspec(D),
                  pl.BlockSpec((1, D, D), lambda b, i: (b, 0, 0)),
                  _wspec2((1, D))],
        out_specs=[_mspec(D), _mspec(D),
                   pl.BlockSpec((1, 3, D), lambda b, i: (b, 0, 0))],
        out_shape=[img(D), img(D), jax.ShapeDtypeStruct((B, 3, D), f32)],
    )(a_arr, xh, v, mproj, r2(p['attn_proj_b']))

    gspec = lambda s: pl.BlockSpec(s, lambda i: (0,) * len(s))
    cof = pl.pallas_call(
        _k5, grid=(1,),
        in_specs=[gspec((B, 3, D)),
                  gspec((D, E)), gspec((1, E)), gspec((D, E)), gspec((1, E)),
                  gspec((D, E)), gspec((1, E)), gspec((D, E)), gspec((1, E))],
        out_specs=gspec((2, B, E)),
        out_shape=jax.ShapeDtypeStruct((2, B, E), f32),
    )(stats,
      jnp.transpose(p['dec_fc0_w']), r2(p['dec_fc0_b']),
      jnp.transpose(p['dec_fc1_w']), r2(p['dec_fc1_b']),
      jnp.transpose(p['det_fc0_w']), r2(p['det_fc0_b']),
      jnp.transpose(p['det_fc1_w']), r2(p['det_fc1_b']))

    def mofe(g, sign, dil, pre):
        import functools
        kfn = functools.partial(_k6, sign=sign, dil=dil)
        return pl.pallas_call(
            kfn, grid=(B, NT),
            in_specs=[_mspec(D), _tspec(D), _bspec(D),
                      pl.BlockSpec(memory_space=pltpu.SMEM),
                      _wspec2((E, 3, 3, D)), _wspec2((E, 1, D)),
                      _wspec2((E, 3, 3, D)), _wspec2((E, 1, D))],
            out_specs=[_mspec(D),
                       pl.BlockSpec((1, 1, D), lambda b, i: (b, 0, 0))],
            out_shape=[img(D), jax.ShapeDtypeStruct((B, 1, D), f32)],
        )(diff, diff, diff, cof[g],
          jnp.transpose(p[pre + '_w1'][:, :, 0], (0, 2, 3, 1)),
          p[pre + '_b1'][:, None, :],
          jnp.transpose(p[pre + '_w2'][:, :, 0], (0, 2, 3, 1)),
          p[pre + '_b2'][:, None, :])

    exp_c, gs_c = mofe(0, 1.0, 1, 'dec')
    exp_t, gs_t = mofe(1, -1.0, 2, 'det')

    eca, eta = pl.pallas_call(
        _k7, grid=(1,),
        in_specs=[gspec((B, 1, D)), gspec((B, 1, D)),
                  gspec((D, 2 * D)), gspec((1, 2 * D)),
                  gspec((2 * D, D)), gspec((1, D)),
                  gspec((D, 2 * D)), gspec((1, 2 * D)),
                  gspec((2 * D, D)), gspec((1, D))],
        out_specs=[gspec((B, 1, D)), gspec((B, 1, D))],
        out_shape=[jax.ShapeDtypeStruct((B, 1, D), f32),
                   jax.ShapeDtypeStruct((B, 1, D), f32)],
    )(gs_c, gs_t,
      jnp.transpose(p['mlp_w1']), r2(p['mlp_b1']),
      jnp.transpose(p['mlp_w2']), r2(p['mlp_b2']),
      jnp.transpose(p['mlp1_w1']), r2(p['mlp1_b1']),
      jnp.transpose(p['mlp1_w2']), r2(p['mlp1_b2']))

    def unet(expa, gate, pre):
        cwt = lambda w: jnp.transpose(w, (2, 3, 1, 0))
        return pl.pallas_call(
            _k8, grid=(B, NT),
            in_specs=[_mspec(D), _tspec(D), _bspec(D),
                      pl.BlockSpec((1, 1, D), lambda b, i: (b, 0, 0)),
                      _wspec2((D, D // 2)), _wspec2((1, D // 2)),
                      _wspec2((3, 3, D // 2, D // 4)), _wspec2((1, D // 4)),
                      _wspec2((3, 3, D // 4, D // 8)), _wspec2((1, D // 8)),
                      _wspec2((3, 3, D // 8, D // 4)), _wspec2((1, D // 4)),
                      _wspec2((3, 3, D // 4, D // 2)), _wspec2((1, D // 2)),
                      _wspec2((D // 2, D)), _wspec2((1, D))],
            out_specs=_mspec(D),
            out_shape=img(D),
        )(expa, expa, expa, gate,
          jnp.transpose(p[pre + '_w1'][:, :, 0, 0]), r2(p[pre + '_b1']),
          cwt(p[pre + '_w2']), r2(p[pre + '_b2']),
          cwt(p[pre + '_w3']), r2(p[pre + '_b3']),
          cwt(p[pre + '_w4']), r2(p[pre + '_b4']),
          cwt(p[pre + '_w5']), r2(p[pre + '_b5']),
          jnp.transpose(p[pre + '_w6'][:, :, 0, 0]), r2(p[pre + '_b6']))

    ecp = unet(exp_c, eca, 'u')
    etp = unet(exp_t, eta, 'u1')

    out = pl.pallas_call(
        _k9, grid=(B, NT),
        in_specs=[_mspec(D), _tspec(D), _bspec(D),
                  _mspec(D), _tspec(D), _bspec(D),
                  _mspec(D), _mspec(D),
                  _wspec2((3, 3, D)), _wspec2((1, D)),
                  _wspec2((3, 3, D)), _wspec2((1, D)),
                  _wspec2((3, 3, D)), _wspec2((1, D)),
                  _wspec2((3, 3, D)), _wspec2((1, D))],
        out_specs=_mspec(D),
        out_shape=img(D),
    )(ecp, ecp, ecp, etp, etp, etp, xb_arr, xh,
      _dwwt(p['X_w']), r2(p['X_b']), _dwwt(p['Y_w']), r2(p['Y_b']),
      _dwwt(p['M_w']), r2(p['M_b']), _dwwt(p['N_w']), r2(p['N_b']))

    return jnp.transpose(out, (0, 3, 1, 2))
